# Initial kernel scaffold; baseline (speedup 1.0000x reference)
import jax
import jax.numpy as jnp
from jax.experimental import pallas as pl


def kernel(x, row_table, col_table, io_table, pair_table):
    raise NotImplementedError("write your pallas kernel here")


# speedup vs baseline: 3.4762x; 3.4762x over previous
import jax
import jax.numpy as jnp
from jax.experimental import pallas as pl


def _body(row_ref, col_ref, io_ref, pair_ref, out_ref, *, H, W, D4):
    g = pl.program_id(0)
    row = row_ref[:H, :]
    col = col_ref[:W, :]
    io = io_ref[pl.ds(g % 2, 1), :]
    pair = pair_ref[pl.ds(g // 2, 1), :]
    out_ref[0, :, :, 0:D4] = jnp.broadcast_to(row[:, None, :], (H, W, D4))
    out_ref[0, :, :, D4:2 * D4] = jnp.broadcast_to(col[None, :, :], (H, W, D4))
    out_ref[0, :, :, 2 * D4:3 * D4] = jnp.broadcast_to(io[None], (H, W, D4))
    out_ref[0, :, :, 3 * D4:4 * D4] = jnp.broadcast_to(pair[None], (H, W, D4))


def kernel(x, row_table, col_table, io_table, pair_table):
    _, G, H, W, D = x.shape
    D4 = row_table.shape[1]
    import functools
    body = functools.partial(_body, H=H, W=W, D4=D4)
    return pl.pallas_call(
        body,
        grid=(G,),
        in_specs=[
            pl.BlockSpec(row_table.shape, lambda g: (0, 0)),
            pl.BlockSpec(col_table.shape, lambda g: (0, 0)),
            pl.BlockSpec(io_table.shape, lambda g: (0, 0)),
            pl.BlockSpec(pair_table.shape, lambda g: (0, 0)),
        ],
        out_specs=pl.BlockSpec((1, H, W, D), lambda g: (g, 0, 0, 0)),
        out_shape=jax.ShapeDtypeStruct((G, H, W, D), x.dtype),
    )(row_table, col_table, io_table, pair_table)
